# Initial kernel scaffold; baseline (speedup 1.0000x reference)
import jax
import jax.numpy as jnp
from jax.experimental import pallas as pl


def kernel(x, token_table, segment_table, ln_scale, ln_bias):
    raise NotImplementedError("write your pallas kernel here")


# speedup vs baseline: 4.6225x; 4.6225x over previous
import functools

import jax
import jax.numpy as jnp
from jax import lax
from jax.experimental import pallas as pl
from jax.experimental.pallas import tpu as pltpu
from jax.experimental.pallas import tpu_sc as plsc

VOCAB = 100000
DIM = 64
B = 4096
S = 200
HALF = 100
NC = 2
NS = 16
NW = NC * NS
SEQ_PER_W = B // NW
LANES = 16
NVREG = DIM // LANES
EPS = 1e-5
_RSQRT_MAGIC = 0x5F3759DF

_GATHER_DNUMS = lax.GatherDimensionNumbers(
    offset_dims=(), collapsed_slice_dims=(0,), start_index_map=(0,))


def _xshuffle(v, k):
    perm = lax.iota(jnp.int32, LANES) ^ k
    return lax.gather(v, perm[:, None], _GATHER_DNUMS, (1,),
                      mode=lax.GatherScatterMode.PROMISE_IN_BOUNDS)


def _rsqrt(t):
    i = lax.bitcast_convert_type(t, jnp.int32)
    y = lax.bitcast_convert_type(_RSQRT_MAGIC - (i >> 1), jnp.float32)
    for _ in range(3):
        y = y * (1.5 - 0.5 * t * y * y)
    return y


def _pos_encoding():
    pos = jnp.arange(S, dtype=jnp.float32)[:, None]
    d = 2.0 * jnp.arange(DIM, dtype=jnp.float32) / DIM
    pe = pos / jnp.power(10000.0, d)
    pe = pe.at[:, 0::2].set(jnp.sin(pe[:, 0::2]))
    pe = pe.at[:, 1::2].set(jnp.cos(pe[:, 1::2]))
    return pe


def _sc_embed(x3, tok, pe, seg2, ln_scale, ln_bias):
    mesh = plsc.VectorSubcoreMesh(core_axis_name="c", subcore_axis_name="s")

    @functools.partial(
        pl.kernel,
        mesh=mesh,
        out_type=jax.ShapeDtypeStruct((B, 2, HALF, DIM), jnp.float32),
        scratch_types=[
            pltpu.VMEM((2, HALF), jnp.int32),
            pltpu.VMEM((2, HALF, DIM), jnp.float32),
            pltpu.VMEM((2, HALF, DIM), jnp.float32),
            pltpu.VMEM((2, HALF, DIM), jnp.float32),
            pltpu.VMEM((2, HALF, DIM), jnp.float32),
            pltpu.VMEM((2, DIM), jnp.float32),
            pltpu.VMEM((DIM,), jnp.float32),
            pltpu.VMEM((DIM,), jnp.float32),
            pltpu.SemaphoreType.DMA,
        ],
        compiler_params=pltpu.CompilerParams(use_tc_tiling_on_sc=False),
    )
    def k(x_hbm, tok_hbm, pe_hbm, seg_hbm, gam_hbm, bet_hbm, out_hbm,
          idx_v, rows_v, obuf, add_v, pe_v, seg_v, gam_v, bet_v, sem):
        wid = lax.axis_index("s") * NC + lax.axis_index("c")
        pltpu.sync_copy(pe_hbm, pe_v)
        pltpu.sync_copy(seg_hbm, seg_v)
        pltpu.sync_copy(gam_hbm, gam_v)
        pltpu.sync_copy(bet_hbm, bet_v)

        def build(i, c):
            h = i // HALF
            r = i % HALF
            srow = jnp.where(i >= S // 2 + 1, 1, 0)
            for g in range(NVREG):
                ds = pl.ds(g * LANES, LANES)
                add_v[h, r, ds] = pe_v[h, r, ds] + seg_v[srow, ds]
            return c
        lax.fori_loop(0, S, build, 0)

        carry0 = (tuple(gam_v[pl.ds(g * LANES, LANES)] for g in range(NVREG))
                  + tuple(bet_v[pl.ds(g * LANES, LANES)] for g in range(NVREG)))

        def seq_body(ch, carry):
            sidx = wid * SEQ_PER_W + ch
            pltpu.sync_copy(x_hbm.at[sidx], idx_v)
            cps = [pltpu.async_copy(tok_hbm.at[idx_v.at[h]], rows_v.at[h], sem)
                   for h in range(2)]
            for cp in cps:
                cp.wait()

            for h in range(2):
                def row(r, c, h=h):
                    v = [rows_v[h, r, pl.ds(g * LANES, LANES)]
                         + add_v[h, r, pl.ds(g * LANES, LANES)]
                         for g in range(NVREG)]
                    sm = (v[0] + v[1]) + (v[2] + v[3])
                    sq = (v[0] * v[0] + v[1] * v[1]) + (v[2] * v[2] + v[3] * v[3])
                    for kk in (1, 2, 4, 8):
                        sm = sm + _xshuffle(sm, kk)
                        sq = sq + _xshuffle(sq, kk)
                    mean = sm * (1.0 / DIM)
                    var = sq * (1.0 / DIM) - mean * mean
                    y = _rsqrt(var + EPS)
                    for g in range(NVREG):
                        obuf[h, r, pl.ds(g * LANES, LANES)] = (
                            (v[g] - mean) * (y * c[g]) + c[NVREG + g])
                    return c
                carry = lax.fori_loop(0, HALF, row, carry)

            pltpu.sync_copy(obuf, out_hbm.at[sidx])
            return carry

        lax.fori_loop(0, SEQ_PER_W, seq_body, carry0)

    return k(x3, tok, pe, seg2, ln_scale, ln_bias)


def kernel(x, token_table, segment_table, ln_scale, ln_bias):
    pe = _pos_encoding().reshape(2, HALF, DIM)
    seg2 = lax.slice_in_dim(segment_table, 0, 2)
    x3 = x.reshape(B, 2, HALF)
    out = _sc_embed(x3, token_table, pe, seg2, ln_scale, ln_bias)
    return out.reshape(B, S, DIM)


# speedup vs baseline: 5.8323x; 1.2617x over previous
import functools

import jax
import jax.numpy as jnp
from jax import lax
from jax.experimental import pallas as pl
from jax.experimental.pallas import tpu as pltpu
from jax.experimental.pallas import tpu_sc as plsc

VOCAB = 100000
DIM = 64
B = 4096
S = 200
HALF = 100
NC = 2
NS = 16
NW = NC * NS
SEQ_PER_W = B // NW
LANES = 16
NVREG = DIM // LANES
EPS = 1e-5
_RSQRT_MAGIC = 0x5F3759DF

_GATHER_DNUMS = lax.GatherDimensionNumbers(
    offset_dims=(), collapsed_slice_dims=(0,), start_index_map=(0,))


def _xshuffle(v, k):
    perm = lax.iota(jnp.int32, LANES) ^ k
    return lax.gather(v, perm[:, None], _GATHER_DNUMS, (1,),
                      mode=lax.GatherScatterMode.PROMISE_IN_BOUNDS)


def _rsqrt(t):
    i = lax.bitcast_convert_type(t, jnp.int32)
    y = lax.bitcast_convert_type(_RSQRT_MAGIC - (i >> 1), jnp.float32)
    for _ in range(3):
        y = y * (1.5 - 0.5 * t * y * y)
    return y


def _pos_encoding():
    pos = jnp.arange(S, dtype=jnp.float32)[:, None]
    d = 2.0 * jnp.arange(DIM, dtype=jnp.float32) / DIM
    pe = pos / jnp.power(10000.0, d)
    pe = pe.at[:, 0::2].set(jnp.sin(pe[:, 0::2]))
    pe = pe.at[:, 1::2].set(jnp.cos(pe[:, 1::2]))
    return pe


def _sc_embed(x3, tok, pe, seg2, ln_scale, ln_bias):
    mesh = plsc.VectorSubcoreMesh(core_axis_name="c", subcore_axis_name="s")

    @functools.partial(
        pl.kernel,
        mesh=mesh,
        out_type=jax.ShapeDtypeStruct((B, 2, HALF, DIM), jnp.float32),
        scratch_types=[
            pltpu.VMEM((2, 2, HALF), jnp.int32),
            pltpu.VMEM((2, 2, HALF, DIM), jnp.float32),
            pltpu.VMEM((2, 2, HALF, DIM), jnp.float32),
            pltpu.VMEM((2, HALF, DIM), jnp.float32),
            pltpu.VMEM((2, HALF, DIM), jnp.float32),
            pltpu.VMEM((2, DIM), jnp.float32),
            pltpu.VMEM((DIM,), jnp.float32),
            pltpu.VMEM((DIM,), jnp.float32),
            pltpu.SemaphoreType.DMA,
            pltpu.SemaphoreType.DMA,
            pltpu.SemaphoreType.DMA,
            pltpu.SemaphoreType.DMA,
            pltpu.SemaphoreType.DMA,
            pltpu.SemaphoreType.DMA,
        ],
        compiler_params=pltpu.CompilerParams(use_tc_tiling_on_sc=False),
    )
    def k(x_hbm, tok_hbm, pe_hbm, seg_hbm, gam_hbm, bet_hbm, out_hbm,
          idx_v, rows_v, obuf, add_v, pe_v, seg_v, gam_v, bet_v,
          gs0, gs1, is0, is1, os0, os1):
        gsem = (gs0, gs1)
        isem = (is0, is1)
        osem = (os0, os1)
        wid = lax.axis_index("s") * NC + lax.axis_index("c")
        base = wid * SEQ_PER_W
        pltpu.sync_copy(pe_hbm, pe_v)
        pltpu.sync_copy(seg_hbm, seg_v)
        pltpu.sync_copy(gam_hbm, gam_v)
        pltpu.sync_copy(bet_hbm, bet_v)

        def build(i, c):
            h = i // HALF
            r = i % HALF
            srow = jnp.where(i >= S // 2 + 1, 1, 0)
            for g in range(NVREG):
                ds = pl.ds(g * LANES, LANES)
                add_v[h, r, ds] = pe_v[h, r, ds] + seg_v[srow, ds]
            return c
        lax.fori_loop(0, S, build, 0)

        def start_gather(buf):
            for h in range(2):
                pltpu.async_copy(tok_hbm.at[idx_v.at[buf, h]],
                                 rows_v.at[buf, h], gsem[buf])

        def wait_gather(buf):
            for h in range(2):
                pltpu.make_async_copy(tok_hbm.at[idx_v.at[buf, h]],
                                      rows_v.at[buf, h], gsem[buf]).wait()

        def compute_seq(buf, carry):
            for h in range(2):
                def row2(r2, c, h=h):
                    for rr in range(2):
                        r = 2 * r2 + rr
                        v = [rows_v[buf, h, r, pl.ds(g * LANES, LANES)]
                             + add_v[h, r, pl.ds(g * LANES, LANES)]
                             for g in range(NVREG)]
                        sm = (v[0] + v[1]) + (v[2] + v[3])
                        sq = ((v[0] * v[0] + v[1] * v[1])
                              + (v[2] * v[2] + v[3] * v[3]))
                        for kk in (1, 2, 4, 8):
                            sm = sm + _xshuffle(sm, kk)
                            sq = sq + _xshuffle(sq, kk)
                        mean = sm * (1.0 / DIM)
                        var = sq * (1.0 / DIM) - mean * mean
                        y = _rsqrt(var + EPS)
                        for g in range(NVREG):
                            obuf[buf, h, r, pl.ds(g * LANES, LANES)] = (
                                (v[g] - mean) * (y * c[g]) + c[NVREG + g])
                    return c
                carry = lax.fori_loop(0, HALF // 2, row2, carry)
            return carry

        pltpu.sync_copy(x_hbm.at[base], idx_v.at[0])
        start_gather(0)
        pltpu.async_copy(x_hbm.at[base + 1], idx_v.at[1], isem[1])

        carry0 = (tuple(gam_v[pl.ds(g * LANES, LANES)] for g in range(NVREG))
                  + tuple(bet_v[pl.ds(g * LANES, LANES)] for g in range(NVREG)))

        def pair(p, carry):
            for cur in range(2):
                nxt = 1 - cur
                i = 2 * p + cur
                sidx = base + i
                wait_gather(cur)

                @pl.when(i + 1 < SEQ_PER_W)
                def _():
                    pltpu.make_async_copy(x_hbm.at[sidx + 1], idx_v.at[nxt],
                                          isem[nxt]).wait()
                    start_gather(nxt)

                @pl.when(i + 2 < SEQ_PER_W)
                def _():
                    pltpu.async_copy(x_hbm.at[sidx + 2], idx_v.at[cur],
                                     isem[cur])

                @pl.when(i >= 2)
                def _():
                    pltpu.make_async_copy(obuf.at[cur], out_hbm.at[sidx - 2],
                                          osem[cur]).wait()

                carry = compute_seq(cur, carry)
                pltpu.async_copy(obuf.at[cur], out_hbm.at[sidx], osem[cur])
            return carry

        lax.fori_loop(0, SEQ_PER_W // 2, pair, carry0)
        for cur in range(2):
            pltpu.make_async_copy(obuf.at[cur], out_hbm.at[base + cur],
                                  osem[cur]).wait()

    return k(x3, tok, pe, seg2, ln_scale, ln_bias)


def kernel(x, token_table, segment_table, ln_scale, ln_bias):
    pe = _pos_encoding().reshape(2, HALF, DIM)
    seg2 = lax.slice_in_dim(segment_table, 0, 2)
    x3 = x.reshape(B, 2, HALF)
    out = _sc_embed(x3, token_table, pe, seg2, ln_scale, ln_bias)
    return out.reshape(B, S, DIM)


# speedup vs baseline: 6.0145x; 1.0312x over previous
import functools

import jax
import jax.numpy as jnp
from jax import lax
from jax.experimental import pallas as pl
from jax.experimental.pallas import tpu as pltpu
from jax.experimental.pallas import tpu_sc as plsc

VOCAB = 100000
DIM = 64
B = 4096
S = 200
HALF = 100
NC = 2
NS = 16
NW = NC * NS
SEQ_PER_W = B // NW
LANES = 16
NVREG = DIM // LANES
EPS = 1e-5
_RSQRT_MAGIC = 0x5F3759DF

_GATHER_DNUMS = lax.GatherDimensionNumbers(
    offset_dims=(), collapsed_slice_dims=(0,), start_index_map=(0,))


def _xshuffle(v, k):
    perm = lax.iota(jnp.int32, LANES) ^ k
    return lax.gather(v, perm[:, None], _GATHER_DNUMS, (1,),
                      mode=lax.GatherScatterMode.PROMISE_IN_BOUNDS)


def _rsqrt(t):
    i = lax.bitcast_convert_type(t, jnp.int32)
    y = lax.bitcast_convert_type(_RSQRT_MAGIC - (i >> 1), jnp.float32)
    for _ in range(2):
        y = y * (1.5 - 0.5 * t * y * y)
    return y


def _pos_encoding():
    pos = jnp.arange(S, dtype=jnp.float32)[:, None]
    d = 2.0 * jnp.arange(DIM, dtype=jnp.float32) / DIM
    pe = pos / jnp.power(10000.0, d)
    pe = pe.at[:, 0::2].set(jnp.sin(pe[:, 0::2]))
    pe = pe.at[:, 1::2].set(jnp.cos(pe[:, 1::2]))
    return pe


def _sc_embed(x3, tok, pe, seg2, ln_scale, ln_bias):
    mesh = plsc.VectorSubcoreMesh(core_axis_name="c", subcore_axis_name="s")

    @functools.partial(
        pl.kernel,
        mesh=mesh,
        out_type=jax.ShapeDtypeStruct((B, 2, HALF, DIM), jnp.float32),
        scratch_types=[
            pltpu.VMEM((2, 2, HALF), jnp.int32),
            pltpu.VMEM((2, 2, HALF, DIM), jnp.float32),
            pltpu.VMEM((2, 2, HALF, DIM), jnp.float32),
            pltpu.VMEM((2, HALF, DIM), jnp.float32),
            pltpu.VMEM((2, HALF, DIM), jnp.float32),
            pltpu.VMEM((2, DIM), jnp.float32),
            pltpu.VMEM((DIM,), jnp.float32),
            pltpu.VMEM((DIM,), jnp.float32),
            pltpu.SemaphoreType.DMA,
            pltpu.SemaphoreType.DMA,
            pltpu.SemaphoreType.DMA,
            pltpu.SemaphoreType.DMA,
            pltpu.SemaphoreType.DMA,
            pltpu.SemaphoreType.DMA,
        ],
        compiler_params=pltpu.CompilerParams(use_tc_tiling_on_sc=False),
    )
    def k(x_hbm, tok_hbm, pe_hbm, seg_hbm, gam_hbm, bet_hbm, out_hbm,
          idx_v, rows_v, obuf, add_v, pe_v, seg_v, gam_v, bet_v,
          gs0, gs1, is0, is1, os0, os1):
        gsem = (gs0, gs1)
        isem = (is0, is1)
        osem = (os0, os1)
        wid = lax.axis_index("s") * NC + lax.axis_index("c")
        base = wid * SEQ_PER_W
        pltpu.sync_copy(pe_hbm, pe_v)
        pltpu.sync_copy(seg_hbm, seg_v)
        pltpu.sync_copy(gam_hbm, gam_v)
        pltpu.sync_copy(bet_hbm, bet_v)

        def build(i, c):
            h = i // HALF
            r = i % HALF
            srow = jnp.where(i >= S // 2 + 1, 1, 0)
            for g in range(NVREG):
                ds = pl.ds(g * LANES, LANES)
                add_v[h, r, ds] = pe_v[h, r, ds] + seg_v[srow, ds]
            return c
        lax.fori_loop(0, S, build, 0)

        def start_gather(buf):
            for h in range(2):
                pltpu.async_copy(tok_hbm.at[idx_v.at[buf, h]],
                                 rows_v.at[buf, h], gsem[buf])

        def wait_gather(buf):
            for h in range(2):
                pltpu.make_async_copy(tok_hbm.at[idx_v.at[buf, h]],
                                      rows_v.at[buf, h], gsem[buf]).wait()

        def compute_seq(buf, carry):
            for h in range(2):
                @plsc.parallel_loop(0, HALF, step=1, unroll=4, carry=carry)
                def row(r, c, h=h):
                    v = [rows_v[buf, h, r, pl.ds(g * LANES, LANES)]
                         + add_v[h, r, pl.ds(g * LANES, LANES)]
                         for g in range(NVREG)]
                    sm = (v[0] + v[1]) + (v[2] + v[3])
                    sq = ((v[0] * v[0] + v[1] * v[1])
                          + (v[2] * v[2] + v[3] * v[3]))
                    for kk in (1, 2, 4, 8):
                        sm = sm + _xshuffle(sm, kk)
                        sq = sq + _xshuffle(sq, kk)
                    mean = sm * (1.0 / DIM)
                    var = sq * (1.0 / DIM) - mean * mean
                    y = _rsqrt(var + EPS)
                    for g in range(NVREG):
                        obuf[buf, h, r, pl.ds(g * LANES, LANES)] = (
                            (v[g] - mean) * (y * c[g]) + c[NVREG + g])
                    return c
                carry = row
            return carry

        pltpu.sync_copy(x_hbm.at[base], idx_v.at[0])
        start_gather(0)
        pltpu.async_copy(x_hbm.at[base + 1], idx_v.at[1], isem[1])

        carry0 = (tuple(gam_v[pl.ds(g * LANES, LANES)] for g in range(NVREG))
                  + tuple(bet_v[pl.ds(g * LANES, LANES)] for g in range(NVREG)))

        def pair(p, carry):
            for cur in range(2):
                nxt = 1 - cur
                i = 2 * p + cur
                sidx = base + i
                wait_gather(cur)

                @pl.when(i + 1 < SEQ_PER_W)
                def _():
                    pltpu.make_async_copy(x_hbm.at[sidx + 1], idx_v.at[nxt],
                                          isem[nxt]).wait()
                    start_gather(nxt)

                @pl.when(i + 2 < SEQ_PER_W)
                def _():
                    pltpu.async_copy(x_hbm.at[sidx + 2], idx_v.at[cur],
                                     isem[cur])

                @pl.when(i >= 2)
                def _():
                    pltpu.make_async_copy(obuf.at[cur], out_hbm.at[sidx - 2],
                                          osem[cur]).wait()

                carry = compute_seq(cur, carry)
                pltpu.async_copy(obuf.at[cur], out_hbm.at[sidx], osem[cur])
            return carry

        lax.fori_loop(0, SEQ_PER_W // 2, pair, carry0)
        for cur in range(2):
            pltpu.make_async_copy(obuf.at[cur], out_hbm.at[base + cur],
                                  osem[cur]).wait()

    return k(x3, tok, pe, seg2, ln_scale, ln_bias)


def kernel(x, token_table, segment_table, ln_scale, ln_bias):
    pe = _pos_encoding().reshape(2, HALF, DIM)
    seg2 = lax.slice_in_dim(segment_table, 0, 2)
    x3 = x.reshape(B, 2, HALF)
    out = _sc_embed(x3, token_table, pe, seg2, ln_scale, ln_bias)
    return out.reshape(B, S, DIM)
